# baseline (speedup 1.0000x reference)
import functools

import jax
import jax.numpy as jnp
from jax import lax
from jax.experimental import pallas as pl
from jax.experimental.pallas import tpu as pltpu
from jax.experimental.pallas import tpu_sc as plsc

VOCAB = 100000
VOCAB_PAD = 100096
EMBED_DIM = 64
BATCH = 4096
MAX_LEN = 200

NUM_WORKERS = 32
D_PER_WORKER = EMBED_DIM // NUM_WORKERS

_MESH = plsc.VectorSubcoreMesh(core_axis_name="c", subcore_axis_name="s")


@functools.partial(
    pl.kernel,
    mesh=_MESH,
    out_type=jax.ShapeDtypeStruct((MAX_LEN, 8, 32, 8, 128), jnp.float32),
    scratch_types=[
        pltpu.VMEM((782, 128), jnp.float32),
        pltpu.VMEM((32, 128), jnp.int32),
        pltpu.VMEM((32, 128), jnp.float32),
        pltpu.SemaphoreType.DMA,
    ],
    compiler_params=pltpu.CompilerParams(
        use_tc_tiling_on_sc=False, needs_layout_passes=False
    ),
)
def _lookup_t(idx_hbm, table_hbm, out_hbm, row_v, idx_v, out_v, sem):
    wid = lax.axis_index("s") * 2 + lax.axis_index("c")

    def d_body(k, carry):
        d = wid * D_PER_WORKER + k
        dp, dq = d // 8, d % 8
        pltpu.sync_copy(table_hbm.at[dp, :, dq], row_v)

        def l_body(l, carry2):
            lp, lq = l // 8, l % 8
            pltpu.sync_copy(idx_hbm.at[lp, :, lq], idx_v)

            def g_body(g, carry3):
                for j in range(8):
                    v16 = idx_v[g, pl.ds(j * 16, 16)]
                    h16 = lax.shift_right_logical(v16, 7)
                    l16 = lax.bitwise_and(v16, 127)
                    out_v[g, pl.ds(j * 16, 16)] = plsc.load_gather(
                        row_v, [h16, l16]
                    )
                return carry3

            lax.fori_loop(0, 32, g_body, 0)
            pltpu.sync_copy(out_v, out_hbm.at[l, dp, :, dq])
            return carry2

        lax.fori_loop(0, MAX_LEN, l_body, 0)
        return carry

    lax.fori_loop(0, D_PER_WORKER, d_body, 0)


def kernel(indices, table):
    idx4 = indices.T.reshape(25, 8, 32, 128).transpose(0, 2, 1, 3)
    table_t = jnp.pad(table.T, ((0, 0), (0, VOCAB_PAD - VOCAB)))
    table4 = table_t.reshape(8, 8, 782, 128).transpose(0, 2, 1, 3)
    out5 = _lookup_t(idx4, table4)
    out = out5.transpose(2, 4, 0, 1, 3).reshape(BATCH, MAX_LEN, EMBED_DIM)
    return out
